# baseline (speedup 1.0000x reference)
import jax
import jax.numpy as jnp
from jax import lax
from jax.experimental import pallas as pl
from jax.experimental.pallas import tpu as pltpu
from jax.experimental.pallas import tpu_sc as plsc

N = 10000
E = 320000
H = 128
HH = H // 2
L = 4

NC = 2
NS = 16
NW = NC * NS
CH = 80

EPS = E // NS
NCH_A = EPS // CH
EPW = E // NW
NCH_G = EPW // CH

_SC_MESH = plsc.VectorSubcoreMesh(core_axis_name="c", subcore_axis_name="s")

_SIG_LOC = [g * 32 + 2 * t + o for g in range(2) for o in range(2)
            for t in range(16)]
_SIGMA = _SIG_LOC + [64 + p for p in _SIG_LOC]


def _sc_agg_body(hh_hbm, e_hbm, ridx_hbm, cidx_hbm, zero_hbm, out_hbm,
                 ridx_v, cidx_v, buf_v, ebuf_v, mbuf_v, agg_sh,
                 gsem0, gsem1, esem0, esem1):
    c = lax.axis_index("c")
    s = lax.axis_index("s")
    @pl.when(s == 0)
    def _():
        pltpu.sync_copy(zero_hbm, agg_sh)
    pltpu.sync_copy(ridx_hbm.at[s], ridx_v)
    pltpu.sync_copy(cidx_hbm.at[s], cidx_v)
    plsc.subcore_barrier()

    h_half = hh_hbm.at[c]
    base = s * EPS
    gsems = (gsem0, gsem1)
    esems = (esem0, esem1)

    def issue(k, b):
        pltpu.async_copy(h_half.at[ridx_v.at[k]], buf_v.at[b], gsems[b])
        pltpu.async_copy(
            e_hbm.at[pl.ds(base + k * CH, CH), pl.ds(c * HH, HH)],
            ebuf_v.at[b], esems[b])

    issue(0, 0)
    issue(1, 1)

    def pair(p, carry):
        for b in range(2):
            k = 2 * p + b
            pltpu.make_async_copy(h_half.at[ridx_v.at[k]], buf_v.at[b],
                                  gsems[b]).wait()
            pltpu.make_async_copy(
                e_hbm.at[pl.ds(base + k * CH, CH), pl.ds(c * HH, HH)],
                ebuf_v.at[b], esems[b]).wait()

            @plsc.parallel_loop(0, CH, step=1, unroll=4)
            def _(i):
                for g in range(HH // 32):
                    hv = buf_v[b, i, pl.ds(g * 32, 32)]
                    a0, a1 = plsc.unpack(
                        hv, format=plsc.PackFormat.INTERLEAVED)
                    s0 = pl.ds(g * 32, 16)
                    s1 = pl.ds(g * 32 + 16, 16)
                    mbuf_v[b, i, s0] = jnp.maximum(a0 + ebuf_v[b, i, s0], 0.0)
                    mbuf_v[b, i, s1] = jnp.maximum(a1 + ebuf_v[b, i, s1], 0.0)
            pltpu.sync_copy(mbuf_v.at[b], agg_sh.at[cidx_v.at[k]], add=True)

            @pl.when(k + 2 < NCH_A)
            def _():
                issue(k + 2, b)
        return carry

    lax.fori_loop(0, NCH_A // 2, pair, 0)
    plsc.subcore_barrier()

    @pl.when(s == 0)
    def _():
        pltpu.sync_copy(agg_sh, out_hbm.at[c])


def _sc_agg(h_halves, e_perm, ridx, cidx, zeros):
    run = pl.kernel(
        _sc_agg_body,
        out_type=jax.ShapeDtypeStruct((NC, N, HH), jnp.float32),
        mesh=_SC_MESH,
        scratch_types=[
            pltpu.VMEM((NCH_A, CH), jnp.int32),
            pltpu.VMEM((NCH_A, CH), jnp.int32),
            pltpu.VMEM((2, CH, HH), jnp.bfloat16),
            pltpu.VMEM((2, CH, HH), jnp.float32),
            pltpu.VMEM((2, CH, HH), jnp.float32),
            pltpu.VMEM_SHARED((N, HH), jnp.float32),
            pltpu.SemaphoreType.DMA,
            pltpu.SemaphoreType.DMA,
            pltpu.SemaphoreType.DMA,
            pltpu.SemaphoreType.DMA,
        ],
        compiler_params=pltpu.CompilerParams(use_tc_tiling_on_sc=False,
                                             needs_layout_passes=False),
    )
    return run(h_halves, e_perm, ridx, cidx, zeros)


def _sc_gather_body(h_hbm, ridx_hbm, cidx_hbm, s_hbm, d_hbm,
                    ridx_v, cidx_v, sbuf_v, dbuf_v,
                    gsem0, gsem1, hsem0, hsem1):
    c = lax.axis_index("c")
    s = lax.axis_index("s")
    wid = c * NS + s
    base = wid * EPW
    pltpu.sync_copy(ridx_hbm.at[wid], ridx_v)
    pltpu.sync_copy(cidx_hbm.at[wid], cidx_v)
    gsems = (gsem0, gsem1)
    hsems = (hsem0, hsem1)

    def issue(k, b):
        pltpu.async_copy(h_hbm.at[ridx_v.at[k]], sbuf_v.at[b], gsems[b])
        pltpu.async_copy(h_hbm.at[cidx_v.at[k]], dbuf_v.at[b], hsems[b])

    issue(0, 0)
    issue(1, 1)

    def emit(k, b):
        pltpu.make_async_copy(h_hbm.at[ridx_v.at[k]], sbuf_v.at[b],
                              gsems[b]).wait()
        pltpu.make_async_copy(h_hbm.at[cidx_v.at[k]], dbuf_v.at[b],
                              hsems[b]).wait()
        pltpu.sync_copy(sbuf_v.at[b], s_hbm.at[pl.ds(base + k * CH, CH)])
        pltpu.sync_copy(dbuf_v.at[b], d_hbm.at[pl.ds(base + k * CH, CH)])

    def pair(p, carry):
        for b in range(2):
            k = 2 * p + b
            emit(k, b)

            @pl.when(k + 2 < NCH_G)
            def _():
                issue(k + 2, b)
        return carry

    lax.fori_loop(0, NCH_G // 2, pair, 0)
    emit(NCH_G - 1, 0)


def _sc_gather(h, ridx, cidx):
    run = pl.kernel(
        _sc_gather_body,
        out_type=(jax.ShapeDtypeStruct((E, H), jnp.float32),
                  jax.ShapeDtypeStruct((E, H), jnp.float32)),
        mesh=_SC_MESH,
        scratch_types=[
            pltpu.VMEM((NCH_G, CH), jnp.int32),
            pltpu.VMEM((NCH_G, CH), jnp.int32),
            pltpu.VMEM((2, CH, H), jnp.float32),
            pltpu.VMEM((2, CH, H), jnp.float32),
            pltpu.SemaphoreType.DMA,
            pltpu.SemaphoreType.DMA,
            pltpu.SemaphoreType.DMA,
            pltpu.SemaphoreType.DMA,
        ],
    )
    return run(h, ridx, cidx)


def _ln(v, g, b):
    mu = jnp.mean(v, axis=-1, keepdims=True)
    var = jnp.mean((v - mu) * (v - mu), axis=-1, keepdims=True)
    return (v - mu) * lax.rsqrt(var + 1e-5) * g + b


def _enc_body(x_ref, w_ref, b_ref, g_ref, be_ref, o_ref, oh_ref):
    v = jnp.dot(x_ref[...], w_ref[...], preferred_element_type=jnp.float32)
    v = jnp.maximum(v + b_ref[...], 0.0)
    v = _ln(v, g_ref[...], be_ref[...])
    o_ref[...] = v
    vh = v.astype(jnp.bfloat16)
    oh_ref[0] = vh[:, :HH]
    oh_ref[1] = vh[:, HH:]


def _tc_encode(x, w, b, g, be, tile):
    n, fin = x.shape
    fout = w.shape[1]
    return pl.pallas_call(
        _enc_body,
        grid=(n // tile,),
        in_specs=[
            pl.BlockSpec((tile, fin), lambda i: (i, 0)),
            pl.BlockSpec((fin, fout), lambda i: (0, 0)),
            pl.BlockSpec((1, fout), lambda i: (0, 0)),
            pl.BlockSpec((1, fout), lambda i: (0, 0)),
            pl.BlockSpec((1, fout), lambda i: (0, 0)),
        ],
        out_specs=(pl.BlockSpec((tile, fout), lambda i: (i, 0)),
                   pl.BlockSpec((NC, tile, HH), lambda i: (0, i, 0))),
        out_shape=(jax.ShapeDtypeStruct((n, fout), jnp.float32),
                   jax.ShapeDtypeStruct((NC, n, HH), jnp.bfloat16)),
    )(x, w, b.reshape(1, -1), g.reshape(1, -1), be.reshape(1, -1))


def _layer_body(h_ref, p_ref, w1_ref, w1s_ref, b1_ref, w2_ref, b2_ref,
                g_ref, be_ref, o_ref, oh_ref):
    h = h_ref[...]
    agg = jnp.concatenate([p_ref[0], p_ref[1]], axis=-1)
    u = (jnp.dot(h, w1_ref[...], preferred_element_type=jnp.float32)
         + jnp.dot(agg, w1s_ref[...], preferred_element_type=jnp.float32))
    u = jnp.maximum(u + b1_ref[...], 0.0)
    v = jnp.dot(u, w2_ref[...], preferred_element_type=jnp.float32) + b2_ref[...]
    v = jnp.maximum(_ln(v, g_ref[...], be_ref[...]), 0.0) + h
    o_ref[...] = v
    vh = v.astype(jnp.bfloat16)
    oh_ref[0] = vh[:, :HH]
    oh_ref[1] = vh[:, HH:]


def _tc_layer(h, parts, w1, w1s, b1, w2, b2, g, be, tile=2000):
    full = lambda i: (0, 0)
    return pl.pallas_call(
        _layer_body,
        grid=(N // tile,),
        in_specs=[
            pl.BlockSpec((tile, H), lambda i: (i, 0)),
            pl.BlockSpec((NC, tile, HH), lambda i: (0, i, 0)),
            pl.BlockSpec((H, H), full),
            pl.BlockSpec((H, H), full),
            pl.BlockSpec((1, H), full),
            pl.BlockSpec((H, H), full),
            pl.BlockSpec((1, H), full),
            pl.BlockSpec((1, H), full),
            pl.BlockSpec((1, H), full),
        ],
        out_specs=(pl.BlockSpec((tile, H), lambda i: (i, 0)),
                   pl.BlockSpec((NC, tile, HH), lambda i: (0, i, 0))),
        out_shape=(jax.ShapeDtypeStruct((N, H), jnp.float32),
                   jax.ShapeDtypeStruct((NC, N, HH), jnp.bfloat16)),
    )(h, parts, w1, w1s, b1.reshape(1, -1), w2, b2.reshape(1, -1),
      g.reshape(1, -1), be.reshape(1, -1))


def _enc_single_body(x_ref, w_ref, b_ref, g_ref, be_ref, o_ref):
    v = jnp.dot(x_ref[...], w_ref[...], preferred_element_type=jnp.float32)
    v = jnp.maximum(v + b_ref[...], 0.0)
    o_ref[...] = _ln(v, g_ref[...], be_ref[...])


def _tc_encode_single(x, w, b, g, be, tile):
    n, fin = x.shape
    fout = w.shape[1]
    return pl.pallas_call(
        _enc_single_body,
        grid=(n // tile,),
        in_specs=[
            pl.BlockSpec((tile, fin), lambda i: (i, 0)),
            pl.BlockSpec((fin, fout), lambda i: (0, 0)),
            pl.BlockSpec((1, fout), lambda i: (0, 0)),
            pl.BlockSpec((1, fout), lambda i: (0, 0)),
            pl.BlockSpec((1, fout), lambda i: (0, 0)),
        ],
        out_specs=pl.BlockSpec((tile, fout), lambda i: (i, 0)),
        out_shape=jax.ShapeDtypeStruct((n, fout), jnp.float32),
    )(x, w, b.reshape(1, -1), g.reshape(1, -1), be.reshape(1, -1))


def _final_body(s_ref, d_ref, e_ref, wa_ref, wb_ref, wc_ref, wd_ref,
                bm1_ref, wm2_ref, bm2_ref, wm3_ref, bm3_ref, o_ref):
    s = s_ref[...]
    d = d_ref[...]
    bf = jnp.bfloat16
    t = jnp.dot(s.astype(bf), wa_ref[...], preferred_element_type=jnp.float32)
    t = t + jnp.dot(d.astype(bf), wb_ref[...],
                    preferred_element_type=jnp.float32)
    t = t + jnp.dot(jnp.abs(s - d).astype(bf), wc_ref[...],
                    preferred_element_type=jnp.float32)
    t = t + jnp.dot(e_ref[...].astype(bf), wd_ref[...],
                    preferred_element_type=jnp.float32)
    t = jnp.maximum(t + bm1_ref[...], 0.0)
    u = jnp.dot(t.astype(bf), wm2_ref[...],
                preferred_element_type=jnp.float32)
    u = jnp.maximum(u + bm2_ref[...], 0.0)
    o_ref[...] = jnp.sum(u * wm3_ref[...], axis=-1, keepdims=True) + bm3_ref[...]


def _tc_final(S, D, e, wa, wb, wc, wd, bm1, Wm2, bm2, Wm3, bm3, tile=2000):
    full = lambda i: (0, 0)
    h2 = Wm2.shape[1]
    out = pl.pallas_call(
        _final_body,
        grid=(E // tile,),
        in_specs=[
            pl.BlockSpec((tile, H), lambda i: (i, 0)),
            pl.BlockSpec((tile, H), lambda i: (i, 0)),
            pl.BlockSpec((tile, H), lambda i: (i, 0)),
            pl.BlockSpec((H, H), full),
            pl.BlockSpec((H, H), full),
            pl.BlockSpec((H, H), full),
            pl.BlockSpec((H, H), full),
            pl.BlockSpec((1, H), full),
            pl.BlockSpec((H, h2), full),
            pl.BlockSpec((1, h2), full),
            pl.BlockSpec((1, h2), full),
            pl.BlockSpec((1, 1), full),
        ],
        out_specs=pl.BlockSpec((tile, 1), lambda i: (i, 0)),
        out_shape=jax.ShapeDtypeStruct((E, 1), jnp.float32),
    )(S, D, e, wa, wb, wc, wd, bm1.reshape(1, -1), Wm2,
      bm2.reshape(1, -1), Wm3.reshape(1, -1), bm3.reshape(1, 1))
    return out[:, 0]


def kernel(x, edge_index, edge_attr, W_ne, b_ne, g_ne, be_ne, W_ee, b_ee,
           g_ee, be_ee, W1, b1, W2, b2, g_ln, be_ln, Wm1, bm1, Wm2, bm2,
           Wm3, bm3):
    row = edge_index[0].astype(jnp.int32)
    col = edge_index[1].astype(jnp.int32)
    ridx_a = row.reshape(NS, NCH_A, CH)
    cidx_a = col.reshape(NS, NCH_A, CH)
    ridx_g = row.reshape(NW, NCH_G, CH)
    cidx_g = col.reshape(NW, NCH_G, CH)
    zeros = jnp.zeros((N, HH), jnp.float32)

    sigma = jnp.array(_SIGMA, dtype=jnp.int32)

    h, hh = _tc_encode(x, W_ne, b_ne, g_ne, be_ne, tile=2000)
    e = _tc_encode_single(edge_attr, W_ee[:, sigma], b_ee[sigma],
                          g_ee[sigma], be_ee[sigma], tile=2000)

    for l in range(L):
        parts = _sc_agg(hh, e, ridx_a, cidx_a, zeros)
        h, hh = _tc_layer(h, parts, W1[l], W1[l][sigma], b1[l], W2[l], b2[l],
                          g_ln[l], be_ln[l])

    S, D = _sc_gather(h, ridx_g, cidx_g)
    bf = jnp.bfloat16
    return _tc_final(S, D, e, Wm1[0:H].astype(bf), Wm1[H:2 * H].astype(bf),
                     Wm1[2 * H:3 * H].astype(bf),
                     Wm1[3 * H:][sigma].astype(bf), bm1, Wm2.astype(bf),
                     bm2, Wm3, bm3)


# speedup vs baseline: 1.0786x; 1.0786x over previous
import jax
import jax.numpy as jnp
from jax import lax
from jax.experimental import pallas as pl
from jax.experimental.pallas import tpu as pltpu
from jax.experimental.pallas import tpu_sc as plsc

N = 10000
E = 320000
H = 128
HH = H // 2
L = 4

NC = 2
NS = 16
NW = NC * NS
CH = 80

EPS = E // NS
NCH_A = EPS // CH
EPW = E // NW
NCH_G = EPW // CH

_SC_MESH = plsc.VectorSubcoreMesh(core_axis_name="c", subcore_axis_name="s")

_SIG_LOC = [g * 32 + 2 * t + o for g in range(2) for o in range(2)
            for t in range(16)]
_SIGMA = _SIG_LOC + [64 + p for p in _SIG_LOC]


_NB = 4
_ZROWS = 632
_ZLAST = N - 15 * _ZROWS


def _sc_agg_body(hh_hbm, e_hbm, ridx_hbm, cidx_hbm, zero_hbm, out_hbm,
                 ridx_v, cidx_v, buf_v, ebuf_v, mbuf_v, agg_sh, *sems):
    gsems, esems, ssems = sems[0:2], sems[2:4], sems[4:6]
    c = lax.axis_index("c")
    s = lax.axis_index("s")
    zoff = pl.multiple_of(s * _ZROWS, 8)

    @pl.when(s < 15)
    def _():
        pltpu.sync_copy(zero_hbm.at[pl.ds(zoff, _ZROWS)],
                        agg_sh.at[pl.ds(zoff, _ZROWS)])

    @pl.when(s == 15)
    def _():
        pltpu.sync_copy(zero_hbm.at[pl.ds(15 * _ZROWS, _ZLAST)],
                        agg_sh.at[pl.ds(15 * _ZROWS, _ZLAST)])

    pltpu.sync_copy(ridx_hbm.at[s], ridx_v)
    pltpu.sync_copy(cidx_hbm.at[s], cidx_v)
    plsc.subcore_barrier()

    h_half = hh_hbm.at[c]
    base = s * EPS

    def issue(k, b):
        pltpu.async_copy(h_half.at[ridx_v.at[k]], buf_v.at[b], gsems[b])
        pltpu.async_copy(
            e_hbm.at[pl.ds(base + k * CH, CH), pl.ds(c * HH, HH)],
            ebuf_v.at[b], esems[b])

    def wait_scatter(k, b):
        pltpu.make_async_copy(mbuf_v.at[b], agg_sh.at[cidx_v.at[k]],
                              ssems[b]).wait()

    def emit(k, b, prefetch):
        @pl.when(k >= 2)
        def _():
            wait_scatter(k - 2, b)
        pltpu.make_async_copy(h_half.at[ridx_v.at[k]], buf_v.at[b],
                              gsems[b]).wait()
        pltpu.make_async_copy(
            e_hbm.at[pl.ds(base + k * CH, CH), pl.ds(c * HH, HH)],
            ebuf_v.at[b], esems[b]).wait()

        @plsc.parallel_loop(0, CH, step=1, unroll=4)
        def _(i):
            for g in range(HH // 32):
                hv = buf_v[b, i, pl.ds(g * 32, 32)]
                a0, a1 = plsc.unpack(hv, format=plsc.PackFormat.INTERLEAVED)
                s0 = pl.ds(g * 32, 16)
                s1 = pl.ds(g * 32 + 16, 16)
                mbuf_v[b, i, s0] = jnp.maximum(a0 + ebuf_v[b, i, s0], 0.0)
                mbuf_v[b, i, s1] = jnp.maximum(a1 + ebuf_v[b, i, s1], 0.0)

        pltpu.async_copy(mbuf_v.at[b], agg_sh.at[cidx_v.at[k]], ssems[b],
                         add=True)
        if prefetch:
            @pl.when(k + 2 < NCH_A)
            def _():
                issue(k + 2, b)

    issue(0, 0)
    issue(1, 1)

    def pair(p, carry):
        for b in range(2):
            emit(2 * p + b, b, True)
        return carry

    lax.fori_loop(0, NCH_A // 2, pair, 0)
    wait_scatter(NCH_A - 2, 0)
    wait_scatter(NCH_A - 1, 1)
    plsc.subcore_barrier()

    @pl.when(s < 15)
    def _():
        pltpu.sync_copy(agg_sh.at[pl.ds(zoff, _ZROWS)],
                        out_hbm.at[c, pl.ds(zoff, _ZROWS)])

    @pl.when(s == 15)
    def _():
        pltpu.sync_copy(agg_sh.at[pl.ds(15 * _ZROWS, _ZLAST)],
                        out_hbm.at[c, pl.ds(15 * _ZROWS, _ZLAST)])


def _sc_agg(h_halves, e_perm, ridx, cidx, zeros):
    run = pl.kernel(
        _sc_agg_body,
        out_type=jax.ShapeDtypeStruct((NC, N, HH), jnp.float32),
        mesh=_SC_MESH,
        scratch_types=[
            pltpu.VMEM((NCH_A, CH), jnp.int32),
            pltpu.VMEM((NCH_A, CH), jnp.int32),
            pltpu.VMEM((2, CH, HH), jnp.bfloat16),
            pltpu.VMEM((2, CH, HH), jnp.float32),
            pltpu.VMEM((2, CH, HH), jnp.float32),
            pltpu.VMEM_SHARED((N, HH), jnp.float32),
        ] + [pltpu.SemaphoreType.DMA] * 6,
        compiler_params=pltpu.CompilerParams(use_tc_tiling_on_sc=False,
                                             needs_layout_passes=False),
    )
    return run(h_halves, e_perm, ridx, cidx, zeros)


def _sc_gather_body(h_hbm, ridx_hbm, cidx_hbm, s_hbm, d_hbm,
                    ridx_v, cidx_v, sbuf_v, dbuf_v, *sems):
    gsems = sems[0:_NB]
    hsems = sems[_NB:2 * _NB]
    osems = sems[2 * _NB:3 * _NB]
    psems = sems[3 * _NB:]
    c = lax.axis_index("c")
    s = lax.axis_index("s")
    wid = c * NS + s
    base = wid * EPW
    pltpu.sync_copy(ridx_hbm.at[wid], ridx_v)
    pltpu.sync_copy(cidx_hbm.at[wid], cidx_v)

    def issue(k, b):
        pltpu.async_copy(h_hbm.at[ridx_v.at[k]], sbuf_v.at[b], gsems[b])
        pltpu.async_copy(h_hbm.at[cidx_v.at[k]], dbuf_v.at[b], hsems[b])

    def wait_out(k, b):
        pltpu.make_async_copy(sbuf_v.at[b], s_hbm.at[pl.ds(base + k * CH, CH)],
                              osems[b]).wait()
        pltpu.make_async_copy(dbuf_v.at[b], d_hbm.at[pl.ds(base + k * CH, CH)],
                              psems[b]).wait()

    def emit(k, b, prefetch):
        pltpu.make_async_copy(h_hbm.at[ridx_v.at[k]], sbuf_v.at[b],
                              gsems[b]).wait()
        pltpu.make_async_copy(h_hbm.at[cidx_v.at[k]], dbuf_v.at[b],
                              hsems[b]).wait()
        pltpu.async_copy(sbuf_v.at[b], s_hbm.at[pl.ds(base + k * CH, CH)],
                         osems[b])
        pltpu.async_copy(dbuf_v.at[b], d_hbm.at[pl.ds(base + k * CH, CH)],
                         psems[b])
        if prefetch:
            nb = (b + 2) % _NB

            @pl.when(k + 2 < NCH_G)
            def _():
                @pl.when(k >= 2)
                def _():
                    wait_out(k - 2, nb)
                issue(k + 2, nb)

    issue(0, 0)
    issue(1, 1)

    def quad(p, carry):
        for b in range(_NB):
            k = _NB * p + b
            emit(k, b, True)
        return carry

    lax.fori_loop(0, NCH_G // _NB, quad, 0)
    emit(NCH_G - 1, (NCH_G - 1) % _NB, False)
    for t in range(_NB):
        kk = NCH_G - _NB + t
        wait_out(kk, kk % _NB)


def _sc_gather(h, ridx, cidx):
    run = pl.kernel(
        _sc_gather_body,
        out_type=(jax.ShapeDtypeStruct((E, H), jnp.float32),
                  jax.ShapeDtypeStruct((E, H), jnp.float32)),
        mesh=_SC_MESH,
        scratch_types=[
            pltpu.VMEM((NCH_G, CH), jnp.int32),
            pltpu.VMEM((NCH_G, CH), jnp.int32),
            pltpu.VMEM((_NB, CH, H), jnp.float32),
            pltpu.VMEM((_NB, CH, H), jnp.float32),
        ] + [pltpu.SemaphoreType.DMA] * (4 * _NB),
    )
    return run(h, ridx, cidx)


def _ln(v, g, b):
    mu = jnp.mean(v, axis=-1, keepdims=True)
    var = jnp.mean((v - mu) * (v - mu), axis=-1, keepdims=True)
    return (v - mu) * lax.rsqrt(var + 1e-5) * g + b


def _enc_body(x_ref, w_ref, b_ref, g_ref, be_ref, o_ref, oh_ref):
    v = jnp.dot(x_ref[...], w_ref[...], preferred_element_type=jnp.float32)
    v = jnp.maximum(v + b_ref[...], 0.0)
    v = _ln(v, g_ref[...], be_ref[...])
    o_ref[...] = v
    vh = v.astype(jnp.bfloat16)
    oh_ref[0] = vh[:, :HH]
    oh_ref[1] = vh[:, HH:]


def _tc_encode(x, w, b, g, be, tile):
    n, fin = x.shape
    fout = w.shape[1]
    return pl.pallas_call(
        _enc_body,
        grid=(n // tile,),
        in_specs=[
            pl.BlockSpec((tile, fin), lambda i: (i, 0)),
            pl.BlockSpec((fin, fout), lambda i: (0, 0)),
            pl.BlockSpec((1, fout), lambda i: (0, 0)),
            pl.BlockSpec((1, fout), lambda i: (0, 0)),
            pl.BlockSpec((1, fout), lambda i: (0, 0)),
        ],
        out_specs=(pl.BlockSpec((tile, fout), lambda i: (i, 0)),
                   pl.BlockSpec((NC, tile, HH), lambda i: (0, i, 0))),
        out_shape=(jax.ShapeDtypeStruct((n, fout), jnp.float32),
                   jax.ShapeDtypeStruct((NC, n, HH), jnp.bfloat16)),
    )(x, w, b.reshape(1, -1), g.reshape(1, -1), be.reshape(1, -1))


def _layer_body(h_ref, p_ref, w1_ref, w1s_ref, b1_ref, w2_ref, b2_ref,
                g_ref, be_ref, o_ref, oh_ref):
    h = h_ref[...]
    agg = jnp.concatenate([p_ref[0], p_ref[1]], axis=-1)
    u = (jnp.dot(h, w1_ref[...], preferred_element_type=jnp.float32)
         + jnp.dot(agg, w1s_ref[...], preferred_element_type=jnp.float32))
    u = jnp.maximum(u + b1_ref[...], 0.0)
    v = jnp.dot(u, w2_ref[...], preferred_element_type=jnp.float32) + b2_ref[...]
    v = jnp.maximum(_ln(v, g_ref[...], be_ref[...]), 0.0) + h
    o_ref[...] = v
    vh = v.astype(jnp.bfloat16)
    oh_ref[0] = vh[:, :HH]
    oh_ref[1] = vh[:, HH:]


def _tc_layer(h, parts, w1, w1s, b1, w2, b2, g, be, tile=2000):
    full = lambda i: (0, 0)
    return pl.pallas_call(
        _layer_body,
        grid=(N // tile,),
        in_specs=[
            pl.BlockSpec((tile, H), lambda i: (i, 0)),
            pl.BlockSpec((NC, tile, HH), lambda i: (0, i, 0)),
            pl.BlockSpec((H, H), full),
            pl.BlockSpec((H, H), full),
            pl.BlockSpec((1, H), full),
            pl.BlockSpec((H, H), full),
            pl.BlockSpec((1, H), full),
            pl.BlockSpec((1, H), full),
            pl.BlockSpec((1, H), full),
        ],
        out_specs=(pl.BlockSpec((tile, H), lambda i: (i, 0)),
                   pl.BlockSpec((NC, tile, HH), lambda i: (0, i, 0))),
        out_shape=(jax.ShapeDtypeStruct((N, H), jnp.float32),
                   jax.ShapeDtypeStruct((NC, N, HH), jnp.bfloat16)),
    )(h, parts, w1, w1s, b1.reshape(1, -1), w2, b2.reshape(1, -1),
      g.reshape(1, -1), be.reshape(1, -1))


def _enc_single_body(x_ref, w_ref, b_ref, g_ref, be_ref, o_ref):
    v = jnp.dot(x_ref[...], w_ref[...], preferred_element_type=jnp.float32)
    v = jnp.maximum(v + b_ref[...], 0.0)
    o_ref[...] = _ln(v, g_ref[...], be_ref[...])


def _tc_encode_single(x, w, b, g, be, tile):
    n, fin = x.shape
    fout = w.shape[1]
    return pl.pallas_call(
        _enc_single_body,
        grid=(n // tile,),
        in_specs=[
            pl.BlockSpec((tile, fin), lambda i: (i, 0)),
            pl.BlockSpec((fin, fout), lambda i: (0, 0)),
            pl.BlockSpec((1, fout), lambda i: (0, 0)),
            pl.BlockSpec((1, fout), lambda i: (0, 0)),
            pl.BlockSpec((1, fout), lambda i: (0, 0)),
        ],
        out_specs=pl.BlockSpec((tile, fout), lambda i: (i, 0)),
        out_shape=jax.ShapeDtypeStruct((n, fout), jnp.float32),
    )(x, w, b.reshape(1, -1), g.reshape(1, -1), be.reshape(1, -1))


def _final_body(s_ref, d_ref, e_ref, wa_ref, wb_ref, wc_ref, wd_ref,
                bm1_ref, wm2_ref, bm2_ref, wm3_ref, bm3_ref, o_ref):
    s = s_ref[...]
    d = d_ref[...]
    t = jnp.dot(s, wa_ref[...], preferred_element_type=jnp.float32)
    t = t + jnp.dot(d, wb_ref[...], preferred_element_type=jnp.float32)
    t = t + jnp.dot(jnp.abs(s - d), wc_ref[...],
                    preferred_element_type=jnp.float32)
    t = t + jnp.dot(e_ref[...], wd_ref[...],
                    preferred_element_type=jnp.float32)
    t = jnp.maximum(t + bm1_ref[...], 0.0)
    u = jnp.dot(t, wm2_ref[...], preferred_element_type=jnp.float32)
    u = jnp.maximum(u + bm2_ref[...], 0.0)
    o_ref[...] = jnp.sum(u * wm3_ref[...], axis=-1, keepdims=True) + bm3_ref[...]


def _tc_final(S, D, e, wa, wb, wc, wd, bm1, Wm2, bm2, Wm3, bm3, tile=2000):
    full = lambda i: (0, 0)
    h2 = Wm2.shape[1]
    out = pl.pallas_call(
        _final_body,
        grid=(E // tile,),
        in_specs=[
            pl.BlockSpec((tile, H), lambda i: (i, 0)),
            pl.BlockSpec((tile, H), lambda i: (i, 0)),
            pl.BlockSpec((tile, H), lambda i: (i, 0)),
            pl.BlockSpec((H, H), full),
            pl.BlockSpec((H, H), full),
            pl.BlockSpec((H, H), full),
            pl.BlockSpec((H, H), full),
            pl.BlockSpec((1, H), full),
            pl.BlockSpec((H, h2), full),
            pl.BlockSpec((1, h2), full),
            pl.BlockSpec((1, h2), full),
            pl.BlockSpec((1, 1), full),
        ],
        out_specs=pl.BlockSpec((tile, 1), lambda i: (i, 0)),
        out_shape=jax.ShapeDtypeStruct((E, 1), jnp.float32),
    )(S, D, e, wa, wb, wc, wd, bm1.reshape(1, -1), Wm2,
      bm2.reshape(1, -1), Wm3.reshape(1, -1), bm3.reshape(1, 1))
    return out[:, 0]


def kernel(x, edge_index, edge_attr, W_ne, b_ne, g_ne, be_ne, W_ee, b_ee,
           g_ee, be_ee, W1, b1, W2, b2, g_ln, be_ln, Wm1, bm1, Wm2, bm2,
           Wm3, bm3):
    row = edge_index[0].astype(jnp.int32)
    col = edge_index[1].astype(jnp.int32)
    ridx_a = row.reshape(NS, NCH_A, CH)
    cidx_a = col.reshape(NS, NCH_A, CH)
    ridx_g = row.reshape(NW, NCH_G, CH)
    cidx_g = col.reshape(NW, NCH_G, CH)
    zeros = jnp.zeros((N, HH), jnp.float32)

    sigma = jnp.array(_SIGMA, dtype=jnp.int32)

    h, hh = _tc_encode(x, W_ne, b_ne, g_ne, be_ne, tile=2000)
    e = _tc_encode_single(edge_attr, W_ee[:, sigma], b_ee[sigma],
                          g_ee[sigma], be_ee[sigma], tile=2000)

    for l in range(L):
        parts = _sc_agg(hh, e, ridx_a, cidx_a, zeros)
        h, hh = _tc_layer(h, parts, W1[l], W1[l][sigma], b1[l], W2[l], b2[l],
                          g_ln[l], be_ln[l])

    S, D = _sc_gather(h, ridx_g, cidx_g)
    return _tc_final(S, D, e, Wm1[0:H], Wm1[H:2 * H], Wm1[2 * H:3 * H],
                     Wm1[3 * H:][sigma], bm1, Wm2, bm2, Wm3, bm3)
